# Initial kernel scaffold; baseline (speedup 1.0000x reference)
import jax
import jax.numpy as jnp
from jax.experimental import pallas as pl


def kernel(h, x, edge_index, edge_attr, params):
    raise NotImplementedError("write your pallas kernel here")


# speedup vs baseline: 2.0618x; 2.0618x over previous
import functools

import jax
import jax.numpy as jnp
from jax import lax
from jax.experimental import pallas as pl
from jax.experimental.pallas import tpu as pltpu
from jax.experimental.pallas import tpu_sc as plsc

NORM_FACTOR = 100.0
XW = 16
XG = 128
NC = 2
NS = 16
C = 80

f32 = jnp.float32


def _mesh():
    return plsc.VectorSubcoreMesh(
        core_axis_name="c", subcore_axis_name="s", num_cores=NC, num_subcores=NS)


def _gather_h(h_tbl, row3d, col3d, with_x, x_tbl=None):
    _, ncw, _ = row3d.shape
    E = NC * NS * ncw * C
    HN = h_tbl.shape[1]

    out_type = [jax.ShapeDtypeStruct((E, HN), f32),
                jax.ShapeDtypeStruct((E, HN), f32)]
    scratch = [pltpu.VMEM((ncw, C), jnp.int32), pltpu.VMEM((ncw, C), jnp.int32),
               pltpu.VMEM((C, HN), f32), pltpu.VMEM((C, HN), f32),
               pltpu.SemaphoreType.DMA, pltpu.SemaphoreType.DMA]
    if with_x:
        out_type += [jax.ShapeDtypeStruct((E, XG), f32),
                     jax.ShapeDtypeStruct((E, XG), f32)]
        scratch += [pltpu.VMEM((C, XG), f32), pltpu.VMEM((C, XG), f32),
                    pltpu.SemaphoreType.DMA, pltpu.SemaphoreType.DMA]

    def body(h_hbm, x_hbm_or_row, *rest):
        if with_x:
            x_hbm = x_hbm_or_row
            (row_hbm, col_hbm, hr_out, hc_out, xr_out, xc_out,
             ri_v, ci_v, hrb, hcb, sem1, sem2, xrb, xcb, sem3, sem4) = rest
        else:
            row_hbm = x_hbm_or_row
            (col_hbm, hr_out, hc_out,
             ri_v, ci_v, hrb, hcb, sem1, sem2) = rest
        c = lax.axis_index("c")
        s = lax.axis_index("s")
        wid = s * NC + c
        base = wid * ncw
        pltpu.sync_copy(row_hbm.at[wid], ri_v)
        pltpu.sync_copy(col_hbm.at[wid], ci_v)

        def step(j, carry):
            eb = (base + j) * C
            cp1 = pltpu.async_copy(h_hbm.at[ri_v.at[j]], hrb, sem1)
            cp2 = pltpu.async_copy(h_hbm.at[ci_v.at[j]], hcb, sem2)
            if with_x:
                cp3 = pltpu.async_copy(x_hbm.at[ri_v.at[j]], xrb, sem3)
                cp4 = pltpu.async_copy(x_hbm.at[ci_v.at[j]], xcb, sem4)
            cp1.wait()
            pltpu.sync_copy(hrb, hr_out.at[pl.ds(eb, C)])
            cp2.wait()
            pltpu.sync_copy(hcb, hc_out.at[pl.ds(eb, C)])
            if with_x:
                cp3.wait()
                pltpu.sync_copy(xrb, xr_out.at[pl.ds(eb, C)])
                cp4.wait()
                pltpu.sync_copy(xcb, xc_out.at[pl.ds(eb, C)])
            return carry

        lax.fori_loop(0, ncw, step, 0)

    args = (h_tbl, x_tbl, row3d, col3d) if with_x else (h_tbl, row3d, col3d)
    return pl.kernel(body, out_type=out_type, mesh=_mesh(),
                     scratch_types=scratch)(*args)


def _scatter_add(vals, row1d, n_pad, zeros_tbl):
    E = row1d.shape[0]
    W = vals.shape[1]
    ncw = E // (NC * NS * C)
    rpt = n_pad // NS

    out_type = jax.ShapeDtypeStruct((NC, n_pad, W), f32)
    scratch = [pltpu.MemorySpace.VMEM_SHARED((n_pad, W), f32),
               pltpu.VMEM((C,), jnp.int32),
               pltpu.VMEM((C, W), f32)]

    def body(vals_hbm, row_hbm, zeros_hbm, out_hbm, acc, idxc, vbuf):
        c = lax.axis_index("c")
        s = lax.axis_index("s")
        pltpu.sync_copy(zeros_hbm.at[pl.ds(s * rpt, rpt)],
                        acc.at[pl.ds(s * rpt, rpt)])
        plsc.subcore_barrier()
        wid = s * NC + c
        base = wid * ncw

        def step(j, carry):
            eb = (base + j) * C
            pltpu.sync_copy(vals_hbm.at[pl.ds(eb, C)], vbuf)
            pltpu.sync_copy(row_hbm.at[pl.ds(eb, C)], idxc)
            pltpu.sync_copy(vbuf, acc.at[idxc], add=True)
            return carry

        lax.fori_loop(0, ncw, step, 0)
        plsc.subcore_barrier()
        pltpu.sync_copy(acc.at[pl.ds(s * rpt, rpt)],
                        out_hbm.at[c].at[pl.ds(s * rpt, rpt)])

    return pl.kernel(body, out_type=out_type, mesh=_mesh(),
                     scratch_types=scratch)(vals, row1d, zeros_tbl)


def _segsum(vals, row1d, n_pad):
    s = jax.ops.segment_sum(vals, row1d, num_segments=n_pad)
    return jnp.stack([s, jnp.zeros_like(s)])


def _silu(v):
    return v * jax.nn.sigmoid(v)


def _edge_mlp1_body(hr, hc, xr, xc, ar, w1h, w1r, w1a, b1, w2, b2, out, rc_out):
    xd = xr[...] - xc[...]
    radial = jnp.sum(xd * xd, axis=1, keepdims=True)
    hcat = jnp.concatenate([hr[...], hc[...]], axis=1)
    pre = jnp.dot(hcat, w1h[...], preferred_element_type=f32)
    pre = pre + radial * w1r[...] + ar[...] * w1a[...] + b1[...]
    a1 = _silu(pre)
    a2 = _silu(jnp.dot(a1, w2[...], preferred_element_type=f32) + b2[...])
    out[...] = a2
    cd = xd[:, :XW] / (jnp.sqrt(radial + 1e-08) + 1.0)
    lane = lax.broadcasted_iota(jnp.int32, (1, XW), 1)
    rc_out[...] = jnp.where(lane < 3, cd, 0.0) + jnp.where(lane == 8, radial, 0.0)


def _edge_mlp2_body(hr, hc, rc, ar, w1h, w1r, w1a, b1, w2, b2, out):
    radial = rc[...][:, 8:9]
    hcat = jnp.concatenate([hr[...], hc[...]], axis=1)
    pre = jnp.dot(hcat, w1h[...], preferred_element_type=f32)
    pre = pre + radial * w1r[...] + ar[...] * w1a[...] + b1[...]
    a1 = _silu(pre)
    a2 = _silu(jnp.dot(a1, w2[...], preferred_element_type=f32) + b2[...])
    out[...] = a2


def _eq_mlp_body(hr, hc, rc, ar, w1h, w1r, w1a, b1, w2, b2, w3, out):
    rcv = rc[...]
    radial = rcv[:, 8:9]
    hcat = jnp.concatenate([hr[...], hc[...]], axis=1)
    pre = jnp.dot(hcat, w1h[...], preferred_element_type=f32)
    pre = pre + radial * w1r[...] + ar[...] * w1a[...] + b1[...]
    a1 = _silu(pre)
    a2 = _silu(jnp.dot(a1, w2[...], preferred_element_type=f32) + b2[...])
    m = jnp.sum(a2 * w3[...], axis=1, keepdims=True)
    lane = lax.broadcasted_iota(jnp.int32, (1, XW), 1)
    out[...] = jnp.where(lane < 3, rcv, 0.0) * m


def _blk(B, W):
    return pl.BlockSpec((B, W), lambda i: (i, 0))


def _full(a):
    return pl.BlockSpec(a.shape, lambda i: (0,) * a.ndim)


def _edge_call1(hr, hc, xr, xc, attr, weights, B):
    E, HN = hr.shape
    in_specs = [_blk(B, HN), _blk(B, HN), _blk(B, XG), _blk(B, XG),
                _blk(B, 1)] + [_full(w) for w in weights]
    return pl.pallas_call(
        _edge_mlp1_body,
        grid=(E // B,),
        in_specs=in_specs,
        out_specs=(_blk(B, HN), _blk(B, XW)),
        out_shape=(jax.ShapeDtypeStruct((E, HN), f32),
                   jax.ShapeDtypeStruct((E, XW), f32)),
    )(hr, hc, xr, xc, attr, *weights)


def _edge_call23(body, hr, hc, rc, attr, weights, out_w, B):
    E, HN = hr.shape
    in_specs = [_blk(B, HN), _blk(B, HN), _blk(B, XW),
                _blk(B, 1)] + [_full(w) for w in weights]
    return pl.pallas_call(
        body,
        grid=(E // B,),
        in_specs=in_specs,
        out_specs=_blk(B, out_w),
        out_shape=jax.ShapeDtypeStruct((E, out_w), f32),
    )(hr, hc, rc, attr, *weights)


def _node_mlp(h, a0, a1, nw1h, nw1a, nb1, nw2, nb2, B):
    N, HN = h.shape

    def body(h_ref, a0_ref, a1_ref, w1h, w1a, b1, w2, b2, out):
        agg = (a0_ref[...] + a1_ref[...]) * (1.0 / NORM_FACTOR)
        pre = (jnp.dot(h_ref[...], w1h[...], preferred_element_type=f32)
               + jnp.dot(agg, w1a[...], preferred_element_type=f32) + b1[...])
        z1 = _silu(pre)
        z = jnp.dot(z1, w2[...], preferred_element_type=f32) + b2[...]
        out[...] = h_ref[...] + z

    return pl.pallas_call(
        body,
        grid=(N // B,),
        in_specs=[_blk(B, HN), _blk(B, HN), _blk(B, HN),
                  _full(nw1h), _full(nw1a), _full(nb1), _full(nw2), _full(nb2)],
        out_specs=_blk(B, HN),
        out_shape=jax.ShapeDtypeStruct((N, HN), f32),
    )(h, a0, a1, nw1h, nw1a, nb1, nw2, nb2)


def _x_update(x, a0, a1, B):
    N = x.shape[0]

    def body(x_ref, a0_ref, a1_ref, out):
        agg = (a0_ref[...] + a1_ref[...]) * (1.0 / NORM_FACTOR)
        out[...] = x_ref[...] + agg[:, 0:3]

    return pl.pallas_call(
        body,
        grid=(N // B,),
        in_specs=[_blk(B, 3), _blk(B, XW), _blk(B, XW)],
        out_specs=_blk(B, 3),
        out_shape=jax.ShapeDtypeStruct((N, 3), f32),
    )(x, a0, a1)


def kernel(h, x, edge_index, edge_attr, params):
    N, HN = h.shape
    E = edge_index.shape[1]
    B_EDGE = 3200
    B_NODE = 2000
    n_pad = -(-N // (8 * NS)) * (8 * NS)

    ncw = E // (NC * NS * C)
    row1d = edge_index[0]
    row3d = edge_index[0].reshape(NC * NS, ncw, C)
    col3d = edge_index[1].reshape(NC * NS, ncw, C)
    xp = jnp.pad(x, ((0, 0), (0, XG - x.shape[1])))
    zeros_h = jnp.zeros((n_pad, HN), f32)
    zeros_x = jnp.zeros((n_pad, XW), f32)

    def edge_weights(w1, b1, w2, b2):
        w1t = w1.T
        return [w1t[:2 * HN], w1t[2 * HN:2 * HN + 1], w1t[2 * HN + 1:2 * HN + 2],
                b1.reshape(1, HN), w2.T, b2.reshape(1, HN)]

    rc = None
    for li, lp in enumerate(params['gcl']):
        ew = edge_weights(lp['ew1'], lp['eb1'], lp['ew2'], lp['eb2'])
        if li == 0:
            hr, hc, xr, xc = _gather_h(h, row3d, col3d, True, x_tbl=xp)
            ef, rc = _edge_call1(hr, hc, xr, xc, edge_attr, ew, B_EDGE)

        else:
            hr, hc = _gather_h(h, row3d, col3d, False)
            ef = _edge_call23(_edge_mlp2_body, hr, hc, rc, edge_attr, ew,
                              HN, B_EDGE)
        agg = _segsum(ef, row1d, n_pad)
        nw1t = lp['nw1'].T
        h = _node_mlp(h, agg[0], agg[1], nw1t[:HN], nw1t[HN:],
                      lp['nb1'].reshape(1, HN), lp['nw2'].T,
                      lp['nb2'].reshape(1, HN), B_NODE)


    eq = params['eq']
    hr, hc = _gather_h(h, row3d, col3d, False)
    ew = edge_weights(eq['w1'], eq['b1'], eq['w2'], eq['b2'])
    trans = _edge_call23(_eq_mlp_body, hr, hc, rc, edge_attr,
                         ew + [eq['w3'].reshape(1, HN)], XW, B_EDGE)
    aggx = _segsum(trans, row1d, n_pad)
    x = _x_update(x, aggx[0], aggx[1], B_NODE)

    return (h, x)
